# Initial kernel scaffold; baseline (speedup 1.0000x reference)
import jax
import jax.numpy as jnp
from jax.experimental import pallas as pl


def kernel(x, edge_index, batch, W1, b1, W2, b2, Wfc, bfc):
    raise NotImplementedError("write your pallas kernel here")


# speedup vs baseline: 15.3346x; 15.3346x over previous
import functools

import jax
import jax.numpy as jnp
from jax import lax
from jax.experimental import pallas as pl
from jax.experimental.pallas import tpu as pltpu
from jax.experimental.pallas import tpu_sc as plsc

N = 10000
E = 320000
D = 128
G = 8
H = 64

NW = 32
CHUNK = 128
NCH = (E + NW * CHUNK - 1) // (NW * CHUNK)
EPAD = NW * NCH * CHUNK
NPAD = 10240
RPT = NPAD // 16
RB = 1000
NB = N // RB

_f32 = jnp.float32


def _sc_mesh():
    return plsc.VectorSubcoreMesh(core_axis_name="c", subcore_axis_name="s")


_SC_PARAMS = pltpu.CompilerParams(use_tc_tiling_on_sc=False)


def _sc_deg(srcs, zeros1d):
    @functools.partial(
        pl.kernel,
        out_type=jax.ShapeDtypeStruct((2, NPAD), _f32),
        mesh=_sc_mesh(),
        compiler_params=_SC_PARAMS,
        scratch_types=[
            pltpu.VMEM((NCH, CHUNK), jnp.int32),
            pltpu.VMEM((CHUNK,), _f32),
            pltpu.VMEM_SHARED((NPAD,), _f32),
        ],
    )
    def k(srcs_hbm, z1_hbm, out_hbm, si_v, ones_v, dacc_sh):
        cid = lax.axis_index("c")
        sid = lax.axis_index("s")
        wid = sid * 2 + cid
        pltpu.sync_copy(srcs_hbm.at[wid], si_v)
        for t in range(CHUNK // 16):
            ones_v[pl.ds(t * 16, 16)] = jnp.ones((16,), _f32)
        pltpu.sync_copy(z1_hbm.at[pl.ds(sid * RPT, RPT)],
                        dacc_sh.at[pl.ds(sid * RPT, RPT)])
        plsc.subcore_barrier()

        def body(j, carry):
            pltpu.sync_copy(ones_v, dacc_sh.at[si_v.at[j]], add=True)
            return carry

        lax.fori_loop(0, NCH, body, 0)
        plsc.subcore_barrier()
        pltpu.sync_copy(dacc_sh.at[pl.ds(sid * RPT, RPT)],
                        out_hbm.at[cid, pl.ds(sid * RPT, RPT)])

    return k(srcs, zeros1d)


def _sc_matvec(w, srcg, dsts, zeros2d):
    @functools.partial(
        pl.kernel,
        out_type=jax.ShapeDtypeStruct((2, NPAD, H), _f32),
        mesh=_sc_mesh(),
        compiler_params=_SC_PARAMS,
        scratch_types=[
            pltpu.VMEM((NCH, CHUNK), jnp.int32),
            pltpu.VMEM((NCH, CHUNK), jnp.int32),
            pltpu.VMEM((CHUNK, H), _f32),
            pltpu.VMEM_SHARED((NPAD, H), _f32),
            pltpu.SemaphoreType.DMA,
        ],
    )
    def k(w_hbm, srcg_hbm, dsts_hbm, z2_hbm, out_hbm, sg_v, ds_v, rows_v,
          acc_sh, sem):
        cid = lax.axis_index("c")
        sid = lax.axis_index("s")
        wid = sid * 2 + cid
        pltpu.sync_copy(srcg_hbm.at[wid], sg_v)
        pltpu.sync_copy(dsts_hbm.at[wid], ds_v)
        pltpu.sync_copy(z2_hbm.at[pl.ds(sid * RPT, RPT)],
                        acc_sh.at[pl.ds(sid * RPT, RPT)])
        plsc.subcore_barrier()

        def body(j, carry):
            pltpu.async_copy(w_hbm.at[sg_v.at[j]], rows_v, sem).wait()
            pltpu.sync_copy(rows_v, acc_sh.at[ds_v.at[j]], add=True)
            return carry

        lax.fori_loop(0, NCH, body, 0)
        plsc.subcore_barrier()
        pltpu.sync_copy(acc_sh.at[pl.ds(sid * RPT, RPT)],
                        out_hbm.at[cid, pl.ds(sid * RPT, RPT)])

    return k(w, srcg, dsts, zeros2d)


def _tc_pre(x, wcat, degpt):
    def body(x_ref, w_ref, dg_ref, xw_ref, dis_ref, wa_ref):
        xw = jnp.dot(x_ref[...], w_ref[...], preferred_element_type=_f32)
        deg = dg_ref[:, 0:1] + dg_ref[:, 1:2]
        dis = jnp.where(deg > 0, lax.rsqrt(jnp.maximum(deg, 1e-12)), 0.0)
        xw_ref[...] = xw
        dis_ref[...] = dis
        wa_ref[...] = dis * xw[:, 2 * H:3 * H]

    return pl.pallas_call(
        body,
        grid=(NB,),
        in_specs=[
            pl.BlockSpec((RB, D), lambda i: (i, 0)),
            pl.BlockSpec((D, 3 * H), lambda i: (0, 0)),
            pl.BlockSpec((RB, 2), lambda i: (i, 0)),
        ],
        out_specs=[
            pl.BlockSpec((RB, 3 * H), lambda i: (i, 0)),
            pl.BlockSpec((RB, 1), lambda i: (i, 0)),
            pl.BlockSpec((RB, H), lambda i: (i, 0)),
        ],
        out_shape=[
            jax.ShapeDtypeStruct((N, 3 * H), _f32),
            jax.ShapeDtypeStruct((N, 1), _f32),
            jax.ShapeDtypeStruct((N, H), _f32),
        ],
    )(x, wcat, degpt)


def _tc_mid(xw, dis, p):
    def body(xw_ref, dis_ref, p_ref, wc_ref):
        ps = p_ref[0] + p_ref[1]
        dis = dis_ref[...]
        wc_ref[...] = dis * (xw_ref[:, H:2 * H] - 2.0 * dis * ps)

    return pl.pallas_call(
        body,
        grid=(NB,),
        in_specs=[
            pl.BlockSpec((RB, 3 * H), lambda i: (i, 0)),
            pl.BlockSpec((RB, 1), lambda i: (i, 0)),
            pl.BlockSpec((2, RB, H), lambda i: (0, i, 0)),
        ],
        out_specs=pl.BlockSpec((RB, H), lambda i: (i, 0)),
        out_shape=jax.ShapeDtypeStruct((N, H), _f32),
    )(xw, dis, p)


def _tc_layer(xw, dis, p, b, w2cat):
    def body(xw_ref, dis_ref, p_ref, b_ref, w2_ref, xw2_ref, wa2_ref):
        ps = p_ref[0] + p_ref[1]
        dis = dis_ref[...]
        h = jnp.maximum(
            xw_ref[:, 0:H] - xw_ref[:, 2 * H:3 * H] - dis * ps + b_ref[...], 0.0)
        xw2 = jnp.dot(h, w2_ref[...], preferred_element_type=_f32)
        xw2_ref[...] = xw2
        wa2_ref[...] = dis * xw2[:, 2 * H:3 * H]

    return pl.pallas_call(
        body,
        grid=(NB,),
        in_specs=[
            pl.BlockSpec((RB, 3 * H), lambda i: (i, 0)),
            pl.BlockSpec((RB, 1), lambda i: (i, 0)),
            pl.BlockSpec((2, RB, H), lambda i: (0, i, 0)),
            pl.BlockSpec((1, H), lambda i: (0, 0)),
            pl.BlockSpec((H, 3 * H), lambda i: (0, 0)),
        ],
        out_specs=[
            pl.BlockSpec((RB, 3 * H), lambda i: (i, 0)),
            pl.BlockSpec((RB, H), lambda i: (i, 0)),
        ],
        out_shape=[
            jax.ShapeDtypeStruct((N, 3 * H), _f32),
            jax.ShapeDtypeStruct((N, H), _f32),
        ],
    )(xw, dis, p, b, w2cat)


def _tc_final(xw2, dis, p, b, batch2d, wfc, bfc):
    def body(xw_ref, dis_ref, p_ref, b_ref, bat_ref, wfc_ref, bfc_ref, out_ref,
             s_sum, s_cnt):
        i = pl.program_id(0)

        @pl.when(i == 0)
        def _():
            s_sum[...] = jnp.zeros((G, H), _f32)
            s_cnt[...] = jnp.zeros((G, 128), _f32)

        ps = p_ref[0] + p_ref[1]
        dis = dis_ref[...]
        h2 = jnp.maximum(
            xw_ref[:, 0:H] - xw_ref[:, 2 * H:3 * H] - dis * ps + b_ref[...], 0.0)
        bat = bat_ref[...][:, 0]
        onehot = (lax.broadcasted_iota(jnp.int32, (G, RB), 0)
                  == bat[None, :]).astype(_f32)
        s_sum[...] += jnp.dot(onehot, h2, preferred_element_type=_f32)
        s_cnt[...] += jnp.broadcast_to(
            jnp.sum(onehot, axis=1)[:, None], (G, 128))

        @pl.when(i == NB - 1)
        def _():
            pooled = s_sum[...] / jnp.maximum(s_cnt[:, 0:1], 1.0)
            out_ref[...] = (jnp.dot(pooled, wfc_ref[...],
                                    preferred_element_type=_f32) + bfc_ref[...])

    return pl.pallas_call(
        body,
        grid=(NB,),
        in_specs=[
            pl.BlockSpec((RB, 3 * H), lambda i: (i, 0)),
            pl.BlockSpec((RB, 1), lambda i: (i, 0)),
            pl.BlockSpec((2, RB, H), lambda i: (0, i, 0)),
            pl.BlockSpec((1, H), lambda i: (0, 0)),
            pl.BlockSpec((RB, 1), lambda i: (i, 0)),
            pl.BlockSpec((H, 1), lambda i: (0, 0)),
            pl.BlockSpec((1, 1), lambda i: (0, 0)),
        ],
        out_specs=pl.BlockSpec((G, 1), lambda i: (0, 0)),
        out_shape=jax.ShapeDtypeStruct((G, 1), _f32),
        scratch_shapes=[
            pltpu.VMEM((G, H), _f32),
            pltpu.VMEM((G, 128), _f32),
        ],
    )(xw2, dis, p, b, batch2d, wfc, bfc)


def kernel(x, edge_index, batch, W1, b1, W2, b2, Wfc, bfc):
    src = edge_index[0].astype(jnp.int32)
    dst = edge_index[1].astype(jnp.int32)

    npad = EPAD - E
    padi = jnp.arange(npad, dtype=jnp.int32)
    srcg = jnp.concatenate([src, padi % N]).reshape(NW, NCH, CHUNK)
    srcs = jnp.concatenate([src, N + (padi % 16)]).reshape(NW, NCH, CHUNK)
    dsts = jnp.concatenate([dst, N + (padi % 16)]).reshape(NW, NCH, CHUNK)

    zeros1d = jnp.zeros((NPAD,), _f32)
    zeros2d = jnp.zeros((NPAD, H), _f32)

    w1cat = jnp.concatenate([W1[0], W1[1], W1[2]], axis=1)
    w2cat = jnp.concatenate([W2[0], W2[1], W2[2]], axis=1)
    b1r = b1.reshape(1, H)
    b2r = b2.reshape(1, H)
    bfcr = bfc.reshape(1, 1)
    batch2d = batch.astype(jnp.int32).reshape(N, 1)

    degp = _sc_deg(srcs, zeros1d)
    degpt = degp.T[:N]

    xw1, dis, wa1 = _tc_pre(x, w1cat, degpt)
    p1 = _sc_matvec(wa1, srcg, dsts, zeros2d)
    wc1 = _tc_mid(xw1, dis, p1[:, :N])
    p2 = _sc_matvec(wc1, srcg, dsts, zeros2d)
    xw2, wa2 = _tc_layer(xw1, dis, p2[:, :N], b1r, w2cat)
    p3 = _sc_matvec(wa2, srcg, dsts, zeros2d)
    wc2 = _tc_mid(xw2, dis, p3[:, :N])
    p4 = _sc_matvec(wc2, srcg, dsts, zeros2d)
    out = _tc_final(xw2, dis, p4[:, :N], b2r, batch2d, wfc=Wfc, bfc=bfcr)
    return out[:, 0]


# speedup vs baseline: 21.5148x; 1.4030x over previous
import functools

import jax
import jax.numpy as jnp
from jax import lax
from jax.experimental import pallas as pl
from jax.experimental.pallas import tpu as pltpu
from jax.experimental.pallas import tpu_sc as plsc

N = 10000
E = 320000
D = 128
G = 8
H = 64

NW = 32
CHUNK = 128
NCH = 80
EPAD = NW * NCH * CHUNK
NPAD = 10240
RPT = NPAD // 16
RB = 1000
NB = N // RB

_f32 = jnp.float32


def _sc_mesh():
    return plsc.VectorSubcoreMesh(core_axis_name="c", subcore_axis_name="s")


_SC_PARAMS = pltpu.CompilerParams(use_tc_tiling_on_sc=False)


def _sc_deg(srcs):
    @functools.partial(
        pl.kernel,
        out_type=jax.ShapeDtypeStruct((2, NPAD), _f32),
        mesh=_sc_mesh(),
        compiler_params=_SC_PARAMS,
        scratch_types=[
            pltpu.VMEM((NCH, CHUNK), jnp.int32),
            pltpu.VMEM((CHUNK,), _f32),
            pltpu.VMEM((RPT,), _f32),
            pltpu.VMEM_SHARED((NPAD,), _f32),
        ],
    )
    def k(srcs_hbm, out_hbm, si_v, ones_v, zb_v, dacc_sh):
        cid = lax.axis_index("c")
        sid = lax.axis_index("s")
        wid = sid * 2 + cid
        pltpu.sync_copy(srcs_hbm.at[wid], si_v)
        for t in range(CHUNK // 16):
            ones_v[pl.ds(t * 16, 16)] = jnp.ones((16,), _f32)

        def zinit(i, carry):
            zb_v[pl.ds(i * 16, 16)] = jnp.zeros((16,), _f32)
            return carry

        lax.fori_loop(0, RPT // 16, zinit, 0)
        pltpu.sync_copy(zb_v, dacc_sh.at[pl.ds(sid * RPT, RPT)])
        plsc.subcore_barrier()

        def body(j, carry):
            pltpu.sync_copy(ones_v, dacc_sh.at[si_v.at[j]], add=True)
            return carry

        lax.fori_loop(0, NCH, body, 0)
        plsc.subcore_barrier()
        pltpu.sync_copy(dacc_sh.at[pl.ds(sid * RPT, RPT)],
                        out_hbm.at[cid, pl.ds(sid * RPT, RPT)])

    return k(srcs)


def _sc_matvec(w, srcg, dsts):
    @functools.partial(
        pl.kernel,
        out_type=jax.ShapeDtypeStruct((2, NPAD, H), _f32),
        mesh=_sc_mesh(),
        compiler_params=_SC_PARAMS,
        scratch_types=[
            pltpu.VMEM((NCH, CHUNK), jnp.int32),
            pltpu.VMEM((NCH, CHUNK), jnp.int32),
            pltpu.VMEM((CHUNK, H), _f32),
            pltpu.VMEM((CHUNK, H), _f32),
            pltpu.VMEM_SHARED((NPAD, H), _f32),
            pltpu.SemaphoreType.DMA,
            pltpu.SemaphoreType.DMA,
        ],
    )
    def k(w_hbm, srcg_hbm, dsts_hbm, out_hbm, sg_v, ds_v, rows0, rows1,
          acc_sh, sem0, sem1):
        cid = lax.axis_index("c")
        sid = lax.axis_index("s")
        wid = sid * 2 + cid
        rows = (rows0, rows1)
        sems = (sem0, sem1)
        pltpu.sync_copy(srcg_hbm.at[wid], sg_v)
        pltpu.sync_copy(dsts_hbm.at[wid], ds_v)

        def zinit(i, carry):
            r = i // (H // 16)
            c = i % (H // 16)
            rows0[r, pl.ds(c * 16, 16)] = jnp.zeros((16,), _f32)
            return carry

        lax.fori_loop(0, CHUNK * (H // 16), zinit, 0)
        for t in range(RPT // CHUNK):
            pltpu.sync_copy(rows0, acc_sh.at[pl.ds(sid * RPT + t * CHUNK, CHUNK)])
        plsc.subcore_barrier()

        pltpu.async_copy(w_hbm.at[sg_v.at[0]], rows0, sem0)
        pltpu.async_copy(w_hbm.at[sg_v.at[1]], rows1, sem1)

        def body(g, carry):
            for b in range(2):
                j = 2 * g + b
                pltpu.make_async_copy(w_hbm.at[sg_v.at[j]], rows[b],
                                      sems[b]).wait()
                pltpu.sync_copy(rows[b], acc_sh.at[ds_v.at[j]], add=True)

                @pl.when(j + 2 < NCH)
                def _():
                    pltpu.async_copy(w_hbm.at[sg_v.at[j + 2]], rows[b], sems[b])

            return carry

        lax.fori_loop(0, NCH // 2, body, 0)
        plsc.subcore_barrier()
        pltpu.sync_copy(acc_sh.at[pl.ds(sid * RPT, RPT)],
                        out_hbm.at[cid, pl.ds(sid * RPT, RPT)])

    return k(w, srcg, dsts)


def _tc_pre(x, wcat, degpt):
    def body(x_ref, w_ref, dg_ref, xw_ref, dis_ref, wa_ref):
        xw = jnp.dot(x_ref[...], w_ref[...], preferred_element_type=_f32)
        deg = dg_ref[:, 0:1] + dg_ref[:, 1:2]
        dis = jnp.where(deg > 0, lax.rsqrt(jnp.maximum(deg, 1e-12)), 0.0)
        xw_ref[...] = xw
        dis_ref[...] = dis
        wa_ref[...] = dis * xw[:, 2 * H:3 * H]

    return pl.pallas_call(
        body,
        grid=(NB,),
        in_specs=[
            pl.BlockSpec((RB, D), lambda i: (i, 0)),
            pl.BlockSpec((D, 3 * H), lambda i: (0, 0)),
            pl.BlockSpec((RB, 2), lambda i: (i, 0)),
        ],
        out_specs=[
            pl.BlockSpec((RB, 3 * H), lambda i: (i, 0)),
            pl.BlockSpec((RB, 1), lambda i: (i, 0)),
            pl.BlockSpec((RB, H), lambda i: (i, 0)),
        ],
        out_shape=[
            jax.ShapeDtypeStruct((N, 3 * H), _f32),
            jax.ShapeDtypeStruct((N, 1), _f32),
            jax.ShapeDtypeStruct((N, H), _f32),
        ],
    )(x, wcat, degpt)


def _tc_mid(xw, dis, p):
    def body(xw_ref, dis_ref, p_ref, wc_ref):
        ps = p_ref[0] + p_ref[1]
        dis = dis_ref[...]
        wc_ref[...] = dis * (xw_ref[:, H:2 * H] - 2.0 * dis * ps)

    return pl.pallas_call(
        body,
        grid=(NB,),
        in_specs=[
            pl.BlockSpec((RB, 3 * H), lambda i: (i, 0)),
            pl.BlockSpec((RB, 1), lambda i: (i, 0)),
            pl.BlockSpec((2, RB, H), lambda i: (0, i, 0)),
        ],
        out_specs=pl.BlockSpec((RB, H), lambda i: (i, 0)),
        out_shape=jax.ShapeDtypeStruct((N, H), _f32),
    )(xw, dis, p)


def _tc_layer(xw, dis, p, b, w2cat):
    def body(xw_ref, dis_ref, p_ref, b_ref, w2_ref, xw2_ref, wa2_ref):
        ps = p_ref[0] + p_ref[1]
        dis = dis_ref[...]
        h = jnp.maximum(
            xw_ref[:, 0:H] - xw_ref[:, 2 * H:3 * H] - dis * ps + b_ref[...], 0.0)
        xw2 = jnp.dot(h, w2_ref[...], preferred_element_type=_f32)
        xw2_ref[...] = xw2
        wa2_ref[...] = dis * xw2[:, 2 * H:3 * H]

    return pl.pallas_call(
        body,
        grid=(NB,),
        in_specs=[
            pl.BlockSpec((RB, 3 * H), lambda i: (i, 0)),
            pl.BlockSpec((RB, 1), lambda i: (i, 0)),
            pl.BlockSpec((2, RB, H), lambda i: (0, i, 0)),
            pl.BlockSpec((1, H), lambda i: (0, 0)),
            pl.BlockSpec((H, 3 * H), lambda i: (0, 0)),
        ],
        out_specs=[
            pl.BlockSpec((RB, 3 * H), lambda i: (i, 0)),
            pl.BlockSpec((RB, H), lambda i: (i, 0)),
        ],
        out_shape=[
            jax.ShapeDtypeStruct((N, 3 * H), _f32),
            jax.ShapeDtypeStruct((N, H), _f32),
        ],
    )(xw, dis, p, b, w2cat)


def _tc_final(xw2, dis, p, b, batch2d, wfc, bfc):
    def body(xw_ref, dis_ref, p_ref, b_ref, bat_ref, wfc_ref, bfc_ref, out_ref,
             s_sum, s_cnt):
        i = pl.program_id(0)

        @pl.when(i == 0)
        def _():
            s_sum[...] = jnp.zeros((G, H), _f32)
            s_cnt[...] = jnp.zeros((G, 128), _f32)

        ps = p_ref[0] + p_ref[1]
        dis = dis_ref[...]
        h2 = jnp.maximum(
            xw_ref[:, 0:H] - xw_ref[:, 2 * H:3 * H] - dis * ps + b_ref[...], 0.0)
        bat = bat_ref[...][:, 0]
        onehot = (lax.broadcasted_iota(jnp.int32, (G, RB), 0)
                  == bat[None, :]).astype(_f32)
        s_sum[...] += jnp.dot(onehot, h2, preferred_element_type=_f32)
        s_cnt[...] += jnp.broadcast_to(
            jnp.sum(onehot, axis=1)[:, None], (G, 128))

        @pl.when(i == NB - 1)
        def _():
            pooled = s_sum[...] / jnp.maximum(s_cnt[:, 0:1], 1.0)
            out_ref[...] = (jnp.dot(pooled, wfc_ref[...],
                                    preferred_element_type=_f32) + bfc_ref[...])

    return pl.pallas_call(
        body,
        grid=(NB,),
        in_specs=[
            pl.BlockSpec((RB, 3 * H), lambda i: (i, 0)),
            pl.BlockSpec((RB, 1), lambda i: (i, 0)),
            pl.BlockSpec((2, RB, H), lambda i: (0, i, 0)),
            pl.BlockSpec((1, H), lambda i: (0, 0)),
            pl.BlockSpec((RB, 1), lambda i: (i, 0)),
            pl.BlockSpec((H, 1), lambda i: (0, 0)),
            pl.BlockSpec((1, 1), lambda i: (0, 0)),
        ],
        out_specs=pl.BlockSpec((G, 1), lambda i: (0, 0)),
        out_shape=jax.ShapeDtypeStruct((G, 1), _f32),
        scratch_shapes=[
            pltpu.VMEM((G, H), _f32),
            pltpu.VMEM((G, 128), _f32),
        ],
    )(xw2, dis, p, b, batch2d, wfc, bfc)


def kernel(x, edge_index, batch, W1, b1, W2, b2, Wfc, bfc):
    src = edge_index[0].astype(jnp.int32)
    dst = edge_index[1].astype(jnp.int32)

    npad = EPAD - E
    padi = jnp.arange(npad, dtype=jnp.int32)
    srcg = jnp.concatenate([src, padi % N]).reshape(NW, NCH, CHUNK)
    srcs = jnp.concatenate([src, N + (padi % 16)]).reshape(NW, NCH, CHUNK)
    dsts = jnp.concatenate([dst, N + (padi % 16)]).reshape(NW, NCH, CHUNK)

    w1cat = jnp.concatenate([W1[0], W1[1], W1[2]], axis=1)
    w2cat = jnp.concatenate([W2[0], W2[1], W2[2]], axis=1)
    b1r = b1.reshape(1, H)
    b2r = b2.reshape(1, H)
    bfcr = bfc.reshape(1, 1)
    batch2d = batch.astype(jnp.int32).reshape(N, 1)

    degp = _sc_deg(srcs)
    degpt = degp.T[:N]

    xw1, dis, wa1 = _tc_pre(x, w1cat, degpt)
    p1 = _sc_matvec(wa1, srcg, dsts)
    wc1 = _tc_mid(xw1, dis, p1[:, :N])
    p2 = _sc_matvec(wc1, srcg, dsts)
    xw2, wa2 = _tc_layer(xw1, dis, p2[:, :N], b1r, w2cat)
    p3 = _sc_matvec(wa2, srcg, dsts)
    wc2 = _tc_mid(xw2, dis, p3[:, :N])
    p4 = _sc_matvec(wc2, srcg, dsts)
    out = _tc_final(xw2, dis, p4[:, :N], b2r, batch2d, wfc=Wfc, bfc=bfcr)
    return out[:, 0]


# speedup vs baseline: 22.8428x; 1.0617x over previous
import functools

import jax
import jax.numpy as jnp
from jax import lax
from jax.experimental import pallas as pl
from jax.experimental.pallas import tpu as pltpu
from jax.experimental.pallas import tpu_sc as plsc

N = 10000
E = 320000
D = 128
G = 8
H = 64

NW = 32
CHUNK = 128
NCH = 80
EPAD = NW * NCH * CHUNK
NPAD = 10240
RPT = NPAD // 16
RB = 1000
NB = N // RB

_f32 = jnp.float32


def _sc_mesh():
    return plsc.VectorSubcoreMesh(core_axis_name="c", subcore_axis_name="s")


_SC_PARAMS = pltpu.CompilerParams(use_tc_tiling_on_sc=False)


def _sc_deg(srcs):
    @functools.partial(
        pl.kernel,
        out_type=jax.ShapeDtypeStruct((2, NPAD), _f32),
        mesh=_sc_mesh(),
        compiler_params=_SC_PARAMS,
        scratch_types=[
            pltpu.VMEM((NCH, CHUNK), jnp.int32),
            pltpu.VMEM((CHUNK,), _f32),
            pltpu.VMEM((RPT,), _f32),
            pltpu.VMEM_SHARED((NPAD,), _f32),
        ],
    )
    def k(srcs_hbm, out_hbm, si_v, ones_v, zb_v, dacc_sh):
        cid = lax.axis_index("c")
        sid = lax.axis_index("s")
        wid = sid * 2 + cid
        pltpu.sync_copy(srcs_hbm.at[wid], si_v)
        for t in range(CHUNK // 16):
            ones_v[pl.ds(t * 16, 16)] = jnp.ones((16,), _f32)

        def zinit(i, carry):
            zb_v[pl.ds(i * 16, 16)] = jnp.zeros((16,), _f32)
            return carry

        lax.fori_loop(0, RPT // 16, zinit, 0)
        pltpu.sync_copy(zb_v, dacc_sh.at[pl.ds(sid * RPT, RPT)])
        plsc.subcore_barrier()

        def body(j, carry):
            pltpu.sync_copy(ones_v, dacc_sh.at[si_v.at[j]], add=True)
            return carry

        lax.fori_loop(0, NCH, body, 0)
        plsc.subcore_barrier()
        pltpu.sync_copy(dacc_sh.at[pl.ds(sid * RPT, RPT)],
                        out_hbm.at[cid, pl.ds(sid * RPT, RPT)])

    return k(srcs)


def _sc_matvec(w, srcg, dsts):
    @functools.partial(
        pl.kernel,
        out_type=jax.ShapeDtypeStruct((2, NPAD, H), _f32),
        mesh=_sc_mesh(),
        compiler_params=_SC_PARAMS,
        scratch_types=[
            pltpu.VMEM((NCH, CHUNK), jnp.int32),
            pltpu.VMEM((NCH, CHUNK), jnp.int32),
            pltpu.VMEM((CHUNK, H), _f32),
            pltpu.VMEM((CHUNK, H), _f32),
            pltpu.VMEM_SHARED((NPAD, H), _f32),
            pltpu.SemaphoreType.DMA,
            pltpu.SemaphoreType.DMA,
        ],
    )
    def k(w_hbm, srcg_hbm, dsts_hbm, out_hbm, sg_v, ds_v, rows0, rows1,
          acc_sh, sem0, sem1):
        cid = lax.axis_index("c")
        sid = lax.axis_index("s")
        wid = sid * 2 + cid
        rows = (rows0, rows1)
        sems = (sem0, sem1)
        pltpu.sync_copy(srcg_hbm.at[wid], sg_v)
        pltpu.sync_copy(dsts_hbm.at[wid], ds_v)

        def zinit(i, carry):
            r = i // (H // 16)
            c = i % (H // 16)
            rows0[r, pl.ds(c * 16, 16)] = jnp.zeros((16,), _f32)
            return carry

        lax.fori_loop(0, CHUNK * (H // 16), zinit, 0)
        for t in range(RPT // CHUNK):
            pltpu.sync_copy(rows0, acc_sh.at[pl.ds(sid * RPT + t * CHUNK, CHUNK)])
        plsc.subcore_barrier()

        pltpu.async_copy(w_hbm.at[sg_v.at[0]], rows0, sem0)
        pltpu.async_copy(w_hbm.at[sg_v.at[1]], rows1, sem1)

        def body(g, carry):
            for b in range(2):
                j = 2 * g + b
                pltpu.make_async_copy(w_hbm.at[sg_v.at[j]], rows[b],
                                      sems[b]).wait()
                pltpu.sync_copy(rows[b], acc_sh.at[ds_v.at[j]], add=True)

                @pl.when(j + 2 < NCH)
                def _():
                    pltpu.async_copy(w_hbm.at[sg_v.at[j + 2]], rows[b], sems[b])

            return carry

        lax.fori_loop(0, NCH // 2, body, 0)
        plsc.subcore_barrier()
        pltpu.sync_copy(acc_sh.at[pl.ds(sid * RPT, RPT)],
                        out_hbm.at[cid, pl.ds(sid * RPT, RPT)])

    return k(w, srcg, dsts)


def _tc_pre(x, wcat, degpt):
    def body(x_ref, w_ref, dg_ref, xw_ref, dis_ref, wa_ref):
        xw = jnp.dot(x_ref[...], w_ref[...], preferred_element_type=_f32)
        deg = dg_ref[:, 0:1] + dg_ref[:, 1:2]
        dis = jnp.where(deg > 0, lax.rsqrt(jnp.maximum(deg, 1e-12)), 0.0)
        xw_ref[...] = xw
        dis_ref[...] = dis
        wa_ref[...] = dis * xw[:, 2 * H:3 * H]

    return pl.pallas_call(
        body,
        grid=(NB,),
        in_specs=[
            pl.BlockSpec((RB, D), lambda i: (i, 0)),
            pl.BlockSpec((D, 3 * H), lambda i: (0, 0)),
            pl.BlockSpec((RB, 2), lambda i: (i, 0)),
        ],
        out_specs=[
            pl.BlockSpec((RB, 3 * H), lambda i: (i, 0)),
            pl.BlockSpec((RB, 1), lambda i: (i, 0)),
            pl.BlockSpec((RB, H), lambda i: (i, 0)),
        ],
        out_shape=[
            jax.ShapeDtypeStruct((N, 3 * H), _f32),
            jax.ShapeDtypeStruct((N, 1), _f32),
            jax.ShapeDtypeStruct((N, H), _f32),
        ],
    )(x, wcat, degpt)


def _tc_mid(xw, dis, p):
    def body(xw_ref, dis_ref, p_ref, wc_ref):
        ps = p_ref[0] + p_ref[1]
        dis = dis_ref[...]
        wc_ref[...] = dis * (xw_ref[:, H:2 * H] - 2.0 * dis * ps)

    return pl.pallas_call(
        body,
        grid=(NB,),
        in_specs=[
            pl.BlockSpec((RB, 3 * H), lambda i: (i, 0)),
            pl.BlockSpec((RB, 1), lambda i: (i, 0)),
            pl.BlockSpec((2, RB, H), lambda i: (0, i, 0)),
        ],
        out_specs=pl.BlockSpec((RB, H), lambda i: (i, 0)),
        out_shape=jax.ShapeDtypeStruct((N, H), _f32),
    )(xw, dis, p)


def _tc_layer(xw, dis, p, b, w2cat):
    def body(xw_ref, dis_ref, p_ref, b_ref, w2_ref, xw2_ref, wa2_ref):
        ps = p_ref[0] + p_ref[1]
        dis = dis_ref[...]
        h = jnp.maximum(
            xw_ref[:, 0:H] - xw_ref[:, 2 * H:3 * H] - dis * ps + b_ref[...], 0.0)
        xw2 = jnp.dot(h, w2_ref[...], preferred_element_type=_f32)
        xw2_ref[...] = xw2
        wa2_ref[...] = dis * xw2[:, 2 * H:3 * H]

    return pl.pallas_call(
        body,
        grid=(NB,),
        in_specs=[
            pl.BlockSpec((RB, 3 * H), lambda i: (i, 0)),
            pl.BlockSpec((RB, 1), lambda i: (i, 0)),
            pl.BlockSpec((2, RB, H), lambda i: (0, i, 0)),
            pl.BlockSpec((1, H), lambda i: (0, 0)),
            pl.BlockSpec((H, 3 * H), lambda i: (0, 0)),
        ],
        out_specs=[
            pl.BlockSpec((RB, 3 * H), lambda i: (i, 0)),
            pl.BlockSpec((RB, H), lambda i: (i, 0)),
        ],
        out_shape=[
            jax.ShapeDtypeStruct((N, 3 * H), _f32),
            jax.ShapeDtypeStruct((N, H), _f32),
        ],
    )(xw, dis, p, b, w2cat)


def _tc_final(xw2, dis, p, b, batch2d, wfc, bfc):
    def body(xw_ref, dis_ref, p_ref, b_ref, bat_ref, wfc_ref, bfc_ref, out_ref,
             s_sum, s_cnt):
        i = pl.program_id(0)

        @pl.when(i == 0)
        def _():
            s_sum[...] = jnp.zeros((G, H), _f32)
            s_cnt[...] = jnp.zeros((G, 128), _f32)

        ps = p_ref[0] + p_ref[1]
        dis = dis_ref[...]
        h2 = jnp.maximum(
            xw_ref[:, 0:H] - xw_ref[:, 2 * H:3 * H] - dis * ps + b_ref[...], 0.0)
        bat = bat_ref[...][:, 0]
        onehot = (lax.broadcasted_iota(jnp.int32, (G, RB), 0)
                  == bat[None, :]).astype(_f32)
        s_sum[...] += jnp.dot(onehot, h2, preferred_element_type=_f32)
        s_cnt[...] += jnp.broadcast_to(
            jnp.sum(onehot, axis=1)[:, None], (G, 128))

        @pl.when(i == NB - 1)
        def _():
            pooled = s_sum[...] / jnp.maximum(s_cnt[:, 0:1], 1.0)
            out_ref[...] = (jnp.dot(pooled, wfc_ref[...],
                                    preferred_element_type=_f32) + bfc_ref[...])

    return pl.pallas_call(
        body,
        grid=(NB,),
        in_specs=[
            pl.BlockSpec((RB, 3 * H), lambda i: (i, 0)),
            pl.BlockSpec((RB, 1), lambda i: (i, 0)),
            pl.BlockSpec((2, RB, H), lambda i: (0, i, 0)),
            pl.BlockSpec((1, H), lambda i: (0, 0)),
            pl.BlockSpec((RB, 1), lambda i: (i, 0)),
            pl.BlockSpec((H, 1), lambda i: (0, 0)),
            pl.BlockSpec((1, 1), lambda i: (0, 0)),
        ],
        out_specs=pl.BlockSpec((G, 1), lambda i: (0, 0)),
        out_shape=jax.ShapeDtypeStruct((G, 1), _f32),
        scratch_shapes=[
            pltpu.VMEM((G, H), _f32),
            pltpu.VMEM((G, 128), _f32),
        ],
    )(xw2, dis, p, b, batch2d, wfc, bfc)


def kernel(x, edge_index, batch, W1, b1, W2, b2, Wfc, bfc):
    src = edge_index[0].astype(jnp.int32)
    dst = edge_index[1].astype(jnp.int32)

    npad = EPAD - E
    padi = jnp.arange(npad, dtype=jnp.int32)
    srcg = jnp.concatenate([src, padi % N]).reshape(NW, NCH, CHUNK)
    srcs = jnp.concatenate([src, N + (padi % 16)]).reshape(NW, NCH, CHUNK)
    dsts = jnp.concatenate([dst, N + (padi % 16)]).reshape(NW, NCH, CHUNK)

    w1cat = jnp.concatenate([W1[0], W1[1], W1[2]], axis=1)
    w2cat = jnp.concatenate([W2[0], W2[1], W2[2]], axis=1)
    b1r = b1.reshape(1, H)
    b2r = b2.reshape(1, H)
    bfcr = bfc.reshape(1, 1)
    batch2d = batch.astype(jnp.int32).reshape(N, 1)

    degp = _sc_deg(srcs)
    degpt = degp.T[:N]

    xw1, dis, wa1 = _tc_pre(x, w1cat, degpt)
    p1 = _sc_matvec(wa1, srcg, dsts)
    wc1 = _tc_mid(xw1, dis, p1)
    p2 = _sc_matvec(wc1, srcg, dsts)
    xw2, wa2 = _tc_layer(xw1, dis, p2, b1r, w2cat)
    p3 = _sc_matvec(wa2, srcg, dsts)
    wc2 = _tc_mid(xw2, dis, p3)
    p4 = _sc_matvec(wc2, srcg, dsts)
    out = _tc_final(xw2, dis, p4, b2r, batch2d, wfc=Wfc, bfc=bfcr)
    return out[:, 0]
